# baseline (speedup 1.0000x reference)
import functools

import jax
import jax.numpy as jnp
from jax import lax
from jax.experimental import pallas as pl
from jax.experimental.pallas import tpu as pltpu
from jax.experimental.pallas import tpu_sc as plsc

N = 10000
E = 320000
D = 128
L = 16
NC = 2
NS = 16
NW = NC * NS
EPW = E // NW
C = 80
CH = EPW // C
GPC = C // L
ROWS_PER_SUB = N // NS

_mesh = functools.partial(
    plsc.VectorSubcoreMesh,
    core_axis_name="c", subcore_axis_name="s", num_cores=NC, num_subcores=NS,
)


def _worker_id():
    return lax.axis_index("s") * NC + lax.axis_index("c")


def _proj_body(src_ref, dst_ref, ws_ref, bs_ref, wd_ref, bd_ref, ps_ref, pd_ref):
    ps_ref[...] = (
        jnp.dot(src_ref[...], ws_ref[...], preferred_element_type=jnp.float32)
        + bs_ref[...]
    )
    pd_ref[...] = (
        jnp.dot(dst_ref[...], wd_ref[...], preferred_element_type=jnp.float32)
        + bd_ref[...]
    )


def _project(src, dst, W_src, b_src, W_dst, b_dst):
    blk = 1000
    grid = N // blk
    return pl.pallas_call(
        _proj_body,
        grid=(grid,),
        in_specs=[
            pl.BlockSpec((blk, D), lambda i: (i, 0)),
            pl.BlockSpec((blk, D), lambda i: (i, 0)),
            pl.BlockSpec((D, D), lambda i: (0, 0)),
            pl.BlockSpec((1, D), lambda i: (0, 0)),
            pl.BlockSpec((D, D), lambda i: (0, 0)),
            pl.BlockSpec((1, D), lambda i: (0, 0)),
        ],
        out_specs=[
            pl.BlockSpec((blk, D), lambda i: (i, 0)),
            pl.BlockSpec((blk, D), lambda i: (i, 0)),
        ],
        out_shape=[
            jax.ShapeDtypeStruct((N, D), jnp.float32),
            jax.ShapeDtypeStruct((N, D), jnp.float32),
        ],
    )(src, dst, W_src, b_src.reshape(1, D), W_dst, b_dst.reshape(1, D))


SNB = 4


def _scores_kernel(psrc, pdst, esrc3, edst3, scores_out, part_out,
                   slab_s, slab_d, rs_bufs, rd_bufs,
                   scores_slab, stage, mz, sems):
    w = _worker_id()
    lane_col = lax.broadcasted_iota(jnp.int32, (L,), 0) * L

    pltpu.sync_copy(esrc3.at[w], slab_s)
    pltpu.sync_copy(edst3.at[w], slab_d)

    mz[pl.ds(0, L)] = jnp.full((L,), -jnp.inf, jnp.float32)
    mz[pl.ds(L, L)] = jnp.zeros((L,), jnp.float32)

    def start_gather(k, b):
        pltpu.async_copy(psrc.at[slab_s.at[k]], rs_bufs[b], sems[b])
        pltpu.async_copy(pdst.at[slab_d.at[k]], rd_bufs[b], sems[b])

    def wait_gather(b):
        pltpu.make_async_copy(psrc.at[slab_s.at[0]], rs_bufs[b], sems[b]).wait()
        pltpu.make_async_copy(pdst.at[slab_d.at[0]], rd_bufs[b], sems[b]).wait()

    def compute(k, b):
        s_rows, d_rows = rs_bufs[b], rd_bufs[b]

        def group(g, carry):
            for i in range(L):
                e = g * L + i
                acc = s_rows[e, pl.ds(0, L)] * d_rows[e, pl.ds(0, L)]
                for j in range(1, D // L):
                    acc = acc + (s_rows[e, pl.ds(j * L, L)]
                                 * d_rows[e, pl.ds(j * L, L)])
                plsc.store_scatter(stage, [lane_col + i], acc)
            sums = stage[pl.ds(0, L)]
            for j in range(1, L):
                sums = sums + stage[pl.ds(j * L, L)]
            scores_slab[pl.ds(k * C + g * L, L)] = sums
            m_old = mz[pl.ds(0, L)]
            z_old = mz[pl.ds(L, L)]
            m_new = jnp.maximum(m_old, sums)
            mz[pl.ds(0, L)] = m_new
            mz[pl.ds(L, L)] = (z_old * jnp.exp(m_old - m_new)
                               + jnp.exp(sums - m_new))
            return carry

        lax.fori_loop(0, GPC, group, 0)

    for b in range(SNB - 1):
        start_gather(b, b)

    def quad(t, carry):
        for j in range(SNB):
            k = SNB * t + j

            def turn(k=k, b=j):
                wait_gather(b)
                compute(k, b)

                @pl.when(k + (SNB - 1) < CH)
                def _():
                    start_gather(k + (SNB - 1), (b + SNB - 1) % SNB)

            if j == 0:
                turn()
            else:
                pl.when(k < CH)(turn)
        return carry

    lax.fori_loop(0, -(-CH // SNB), quad, 0)

    m16 = mz[pl.ds(0, L)]
    z16 = mz[pl.ds(L, L)]
    m_loc = jnp.max(m16)
    bm = jnp.broadcast_to(m_loc, (L,))
    z_loc = jnp.sum(z16 * jnp.exp(m16 - bm))
    mz[pl.ds(0, L)] = bm
    mz[pl.ds(L, L)] = jnp.broadcast_to(z_loc, (L,))

    pltpu.sync_copy(scores_slab, scores_out.at[pl.ds(w * EPW, EPW)])
    pltpu.sync_copy(mz, part_out.at[w])


def _edge_scores(psrc, pdst, esrc3, edst3):
    def body(psrc_r, pdst_r, esrc_r, edst_r, scores_r, part_r,
             slab_s, slab_d, rs0, rs1, rs2, rs3, rd0, rd1, rd2, rd3,
             scores_slab, stage, mz, m0, m1, m2, m3):
        _scores_kernel(psrc_r, pdst_r, esrc_r, edst_r, scores_r, part_r,
                       slab_s, slab_d, [rs0, rs1, rs2, rs3],
                       [rd0, rd1, rd2, rd3], scores_slab, stage, mz,
                       [m0, m1, m2, m3])

    return pl.kernel(
        body,
        out_type=[
            jax.ShapeDtypeStruct((E,), jnp.float32),
            jax.ShapeDtypeStruct((NW, 2 * L), jnp.float32),
        ],
        mesh=_mesh(),
        compiler_params=pltpu.CompilerParams(needs_layout_passes=False),
        scratch_types=[
            pltpu.VMEM((CH, C), jnp.int32),
            pltpu.VMEM((CH, C), jnp.int32),
            pltpu.VMEM((C, D), jnp.float32),
            pltpu.VMEM((C, D), jnp.float32),
            pltpu.VMEM((C, D), jnp.float32),
            pltpu.VMEM((C, D), jnp.float32),
            pltpu.VMEM((C, D), jnp.float32),
            pltpu.VMEM((C, D), jnp.float32),
            pltpu.VMEM((C, D), jnp.float32),
            pltpu.VMEM((C, D), jnp.float32),
            pltpu.VMEM((EPW,), jnp.float32),
            pltpu.VMEM((L * L,), jnp.float32),
            pltpu.VMEM((2 * L,), jnp.float32),
            pltpu.SemaphoreType.DMA,
            pltpu.SemaphoreType.DMA,
            pltpu.SemaphoreType.DMA,
            pltpu.SemaphoreType.DMA,
        ],
    )(psrc, pdst, esrc3, edst3)


RB = 3
SB = 3


def _agg_kernel(psrc, esrc2, edst3, scores, parts, zeros_blk, part_out,
                slab_d, rows_bufs, idx_bufs, sc_bufs, pbuf, acc_shared,
                gsems, ssems, smsems):
    c = lax.axis_index("c")
    s = lax.axis_index("s")
    w = s * NC + c

    pltpu.sync_copy(zeros_blk, acc_shared.at[pl.ds(s * ROWS_PER_SUB, ROWS_PER_SUB)])
    pltpu.sync_copy(edst3.at[w], slab_d)
    pltpu.sync_copy(parts, pbuf)
    m16 = pbuf[0, pl.ds(0, L)]
    for q in range(1, NW):
        m16 = jnp.maximum(m16, pbuf[q, pl.ds(0, L)])
    z16 = pbuf[0, pl.ds(L, L)] * jnp.exp(pbuf[0, pl.ds(0, L)] - m16)
    for q in range(1, NW):
        z16 = z16 + pbuf[q, pl.ds(L, L)] * jnp.exp(pbuf[q, pl.ds(0, L)] - m16)
    winv = 1.0 / z16
    plsc.subcore_barrier()

    def start_small(k, sb):
        pltpu.async_copy(esrc2.at[w * CH + k], idx_bufs[sb], smsems[sb])
        pltpu.async_copy(scores.at[pl.ds(w * EPW + k * C, C)], sc_bufs[sb],
                         smsems[sb])

    def wait_small(sb):
        pltpu.make_async_copy(esrc2.at[0], idx_bufs[sb], smsems[sb]).wait()
        pltpu.make_async_copy(scores.at[pl.ds(0, C)], sc_bufs[sb],
                              smsems[sb]).wait()

    def start_gather(sb, rb):
        pltpu.async_copy(psrc.at[idx_bufs[sb]], rows_bufs[rb], gsems[rb])

    def wait_gather(rb):
        pltpu.make_async_copy(psrc.at[idx_bufs[0]], rows_bufs[rb], gsems[rb]).wait()

    def start_scatter(k, rb):
        pltpu.async_copy(rows_bufs[rb], acc_shared.at[slab_d.at[k]], ssems[rb],
                         add=True)

    def wait_scatter(rb):
        pltpu.make_async_copy(rows_bufs[rb], acc_shared.at[slab_d.at[0]],
                              ssems[rb]).wait()

    def compute(sb, rb):
        rows = rows_bufs[rb]
        sc_c = sc_bufs[sb]

        def group(g, carry):
            for i in range(L):
                e = g * L + i
                sc = plsc.load_gather(sc_c, [jnp.broadcast_to(e, (L,))])
                we = jnp.exp(sc - m16) * winv
                for j in range(D // L):
                    rows[e, pl.ds(j * L, L)] = rows[e, pl.ds(j * L, L)] * we
            return carry

        lax.fori_loop(0, GPC, group, 0)

    start_small(0, 0)
    start_small(1, 1)
    wait_small(0)
    start_gather(0, 0)

    def trip(t, carry):
        for j in range(RB):
            k = RB * t + j
            rb = j
            nrb = (j + 1) % RB
            nsb = (j + 1) % SB
            psb = (j + 2) % SB

            def turn(k=k, rb=rb, nrb=nrb, nsb=nsb, psb=psb, j=j):
                @pl.when(k + 1 < CH)
                def _():
                    wait_small(nsb)

                    @pl.when(k >= 2)
                    def _():
                        wait_scatter(nrb)

                    start_gather(nsb, nrb)

                wait_gather(rb)
                compute(j % SB, rb)
                start_scatter(k, rb)

                @pl.when(k + 2 < CH)
                def _():
                    start_small(k + 2, psb)

            if j == 0:
                turn()
            else:
                pl.when(k < CH)(turn)
        return carry

    lax.fori_loop(0, -(-CH // RB), trip, 0)

    for rb in range(RB):
        wait_scatter(rb)
    plsc.subcore_barrier()

    r0 = s * ROWS_PER_SUB
    pltpu.sync_copy(
        acc_shared.at[pl.ds(r0, ROWS_PER_SUB)],
        part_out.at[c, pl.ds(r0, ROWS_PER_SUB)],
    )


def _aggregate(psrc, esrc2, edst3, scores, parts):
    zeros_blk = jnp.zeros((ROWS_PER_SUB, D), jnp.float32)

    def body(psrc_r, esrc_r, edst_r, scores_r, parts_r, zeros_r, out_r,
             slab_d, r0, r1, r2, i0, i1, i2, s0, s1, s2, pbuf, acc_shared,
             g0, g1, g2, t0, t1, t2, m0, m1, m2):
        _agg_kernel(psrc_r, esrc_r, edst_r, scores_r, parts_r, zeros_r, out_r,
                    slab_d, [r0, r1, r2], [i0, i1, i2], [s0, s1, s2],
                    pbuf, acc_shared, [g0, g1, g2], [t0, t1, t2],
                    [m0, m1, m2])

    return pl.kernel(
        body,
        out_type=jax.ShapeDtypeStruct((NC, N, D), jnp.float32),
        mesh=_mesh(),
        compiler_params=pltpu.CompilerParams(
            needs_layout_passes=False, use_tc_tiling_on_sc=False),
        scratch_types=[
            pltpu.VMEM((CH, C), jnp.int32),
            pltpu.VMEM((C, D), jnp.float32),
            pltpu.VMEM((C, D), jnp.float32),
            pltpu.VMEM((C, D), jnp.float32),
            pltpu.VMEM((C,), jnp.int32),
            pltpu.VMEM((C,), jnp.int32),
            pltpu.VMEM((C,), jnp.int32),
            pltpu.VMEM((C,), jnp.float32),
            pltpu.VMEM((C,), jnp.float32),
            pltpu.VMEM((C,), jnp.float32),
            pltpu.VMEM((NW, 2 * L), jnp.float32),
            pltpu.VMEM_SHARED((N, D), jnp.float32),
            pltpu.SemaphoreType.DMA,
            pltpu.SemaphoreType.DMA,
            pltpu.SemaphoreType.DMA,
            pltpu.SemaphoreType.DMA,
            pltpu.SemaphoreType.DMA,
            pltpu.SemaphoreType.DMA,
            pltpu.SemaphoreType.DMA,
            pltpu.SemaphoreType.DMA,
            pltpu.SemaphoreType.DMA,
        ],
    )(psrc, esrc2, edst3, scores, parts, zeros_blk)


def _combine_body(p_ref, o_ref):
    o_ref[...] = p_ref[0] + p_ref[1]


def _combine(partials):
    return pl.pallas_call(
        _combine_body,
        out_shape=jax.ShapeDtypeStruct((N, D), jnp.float32),
    )(partials)


def kernel(src, dst, edge_index, W_src, b_src, W_dst, b_dst):
    esrc3 = edge_index[0].reshape(NW, CH, C)
    esrc2 = edge_index[0].reshape(NW * CH, C)
    edst3 = edge_index[1].reshape(NW, CH, C)
    psrc, pdst = _project(src, dst, W_src, b_src, W_dst, b_dst)
    scores, parts = _edge_scores(psrc, pdst, esrc3, edst3)
    partials = _aggregate(psrc, esrc2, edst3, scores, parts)
    return _combine(partials)


# speedup vs baseline: 1.1968x; 1.1968x over previous
import functools

import jax
import jax.numpy as jnp
from jax import lax
from jax.experimental import pallas as pl
from jax.experimental.pallas import tpu as pltpu
from jax.experimental.pallas import tpu_sc as plsc

N = 10000
E = 320000
D = 128
L = 16
NC = 2
NS = 16
NW = NC * NS
EPW = E // NW
C = 80
CH = EPW // C
GPC = C // L
ROWS_PER_SUB = N // NS

_mesh = functools.partial(
    plsc.VectorSubcoreMesh,
    core_axis_name="c", subcore_axis_name="s", num_cores=NC, num_subcores=NS,
)


def _worker_id():
    return lax.axis_index("s") * NC + lax.axis_index("c")


def _proj_body(src_ref, dst_ref, ws_ref, bs_ref, wd_ref, bd_ref, ps_ref, pd_ref):
    ps_ref[...] = (
        jnp.dot(src_ref[...], ws_ref[...], preferred_element_type=jnp.float32)
        + bs_ref[...]
    )
    pd_ref[...] = (
        jnp.dot(dst_ref[...], wd_ref[...], preferred_element_type=jnp.float32)
        + bd_ref[...]
    )


def _project(src, dst, W_src, b_src, W_dst, b_dst):
    blk = 1000
    grid = N // blk
    return pl.pallas_call(
        _proj_body,
        grid=(grid,),
        in_specs=[
            pl.BlockSpec((blk, D), lambda i: (i, 0)),
            pl.BlockSpec((blk, D), lambda i: (i, 0)),
            pl.BlockSpec((D, D), lambda i: (0, 0)),
            pl.BlockSpec((1, D), lambda i: (0, 0)),
            pl.BlockSpec((D, D), lambda i: (0, 0)),
            pl.BlockSpec((1, D), lambda i: (0, 0)),
        ],
        out_specs=[
            pl.BlockSpec((blk, D), lambda i: (i, 0)),
            pl.BlockSpec((blk, D), lambda i: (i, 0)),
        ],
        out_shape=[
            jax.ShapeDtypeStruct((N, D), jnp.float32),
            jax.ShapeDtypeStruct((N, D), jnp.float32),
        ],
    )(src, dst, W_src, b_src.reshape(1, D), W_dst, b_dst.reshape(1, D))


SNB = 4


def _scores_kernel(psrc, pdst, esrc3, edst3, scores_out, part_out,
                   slab_s, slab_d, rs_bufs, rd_bufs,
                   scores_slab, stage, mz, sems):
    w = _worker_id()
    lane_col = lax.broadcasted_iota(jnp.int32, (L,), 0) * L

    pltpu.sync_copy(esrc3.at[w], slab_s)
    pltpu.sync_copy(edst3.at[w], slab_d)

    mz[pl.ds(0, L)] = jnp.full((L,), -jnp.inf, jnp.float32)
    mz[pl.ds(L, L)] = jnp.zeros((L,), jnp.float32)

    def start_gather(k, b):
        pltpu.async_copy(psrc.at[slab_s.at[k]], rs_bufs[b], sems[b])
        pltpu.async_copy(pdst.at[slab_d.at[k]], rd_bufs[b], sems[b])

    def wait_gather(b):
        pltpu.make_async_copy(psrc.at[slab_s.at[0]], rs_bufs[b], sems[b]).wait()
        pltpu.make_async_copy(pdst.at[slab_d.at[0]], rd_bufs[b], sems[b]).wait()

    def compute(k, b):
        s_rows, d_rows = rs_bufs[b], rd_bufs[b]

        def group(g, carry):
            for i in range(L):
                e = g * L + i
                acc = s_rows[e, pl.ds(0, L)] * d_rows[e, pl.ds(0, L)]
                for j in range(1, D // L):
                    acc = acc + (s_rows[e, pl.ds(j * L, L)]
                                 * d_rows[e, pl.ds(j * L, L)])
                plsc.store_scatter(stage, [lane_col + i], acc)
            sums = stage[pl.ds(0, L)]
            for j in range(1, L):
                sums = sums + stage[pl.ds(j * L, L)]
            scores_slab[pl.ds(k * C + g * L, L)] = sums
            m_old = mz[pl.ds(0, L)]
            z_old = mz[pl.ds(L, L)]
            m_new = jnp.maximum(m_old, sums)
            mz[pl.ds(0, L)] = m_new
            mz[pl.ds(L, L)] = (z_old * jnp.exp(m_old - m_new)
                               + jnp.exp(sums - m_new))
            return carry

        lax.fori_loop(0, GPC, group, 0)

    for b in range(SNB - 1):
        start_gather(b, b)

    def quad(t, carry):
        for j in range(SNB):
            k = SNB * t + j

            def turn(k=k, b=j):
                wait_gather(b)
                compute(k, b)

                @pl.when(k + (SNB - 1) < CH)
                def _():
                    start_gather(k + (SNB - 1), (b + SNB - 1) % SNB)

            if j == 0:
                turn()
            else:
                pl.when(k < CH)(turn)
        return carry

    lax.fori_loop(0, -(-CH // SNB), quad, 0)

    m16 = mz[pl.ds(0, L)]
    z16 = mz[pl.ds(L, L)]
    m_loc = jnp.max(m16)
    bm = jnp.broadcast_to(m_loc, (L,))
    z_loc = jnp.sum(z16 * jnp.exp(m16 - bm))
    mz[pl.ds(0, L)] = bm
    mz[pl.ds(L, L)] = jnp.broadcast_to(z_loc, (L,))

    pltpu.sync_copy(scores_slab, scores_out.at[pl.ds(w * EPW, EPW)])
    pltpu.sync_copy(mz, part_out.at[w])


def _edge_scores(psrc, pdst, esrc3, edst3):
    def body(psrc_r, pdst_r, esrc_r, edst_r, scores_r, part_r,
             slab_s, slab_d, rs0, rs1, rs2, rs3, rd0, rd1, rd2, rd3,
             scores_slab, stage, mz, m0, m1, m2, m3):
        _scores_kernel(psrc_r, pdst_r, esrc_r, edst_r, scores_r, part_r,
                       slab_s, slab_d, [rs0, rs1, rs2, rs3],
                       [rd0, rd1, rd2, rd3], scores_slab, stage, mz,
                       [m0, m1, m2, m3])

    return pl.kernel(
        body,
        out_type=[
            jax.ShapeDtypeStruct((E,), jnp.float32),
            jax.ShapeDtypeStruct((NW, 2 * L), jnp.float32),
        ],
        mesh=_mesh(),
        compiler_params=pltpu.CompilerParams(needs_layout_passes=False),
        scratch_types=[
            pltpu.VMEM((CH, C), jnp.int32),
            pltpu.VMEM((CH, C), jnp.int32),
            pltpu.VMEM((C, D), jnp.float32),
            pltpu.VMEM((C, D), jnp.float32),
            pltpu.VMEM((C, D), jnp.float32),
            pltpu.VMEM((C, D), jnp.float32),
            pltpu.VMEM((C, D), jnp.float32),
            pltpu.VMEM((C, D), jnp.float32),
            pltpu.VMEM((C, D), jnp.float32),
            pltpu.VMEM((C, D), jnp.float32),
            pltpu.VMEM((EPW,), jnp.float32),
            pltpu.VMEM((L * L,), jnp.float32),
            pltpu.VMEM((2 * L,), jnp.float32),
            pltpu.SemaphoreType.DMA,
            pltpu.SemaphoreType.DMA,
            pltpu.SemaphoreType.DMA,
            pltpu.SemaphoreType.DMA,
        ],
    )(psrc, pdst, esrc3, edst3)


RB = 3
SB = 3


def _agg_kernel(psrc, esrc2, edst3, scores, parts, zeros_blk, part_out,
                slab_d, rows_bufs, idx_bufs, sc_bufs, pbuf, acc_shared,
                gsems, ssems, smsems):
    c = lax.axis_index("c")
    s = lax.axis_index("s")
    w = s * NC + c

    pltpu.sync_copy(zeros_blk, acc_shared.at[pl.ds(s * ROWS_PER_SUB, ROWS_PER_SUB)])
    pltpu.sync_copy(edst3.at[w], slab_d)
    pltpu.sync_copy(parts, pbuf)
    m16 = pbuf[0, pl.ds(0, L)]
    for q in range(1, NW):
        m16 = jnp.maximum(m16, pbuf[q, pl.ds(0, L)])
    z16 = pbuf[0, pl.ds(L, L)] * jnp.exp(pbuf[0, pl.ds(0, L)] - m16)
    for q in range(1, NW):
        z16 = z16 + pbuf[q, pl.ds(L, L)] * jnp.exp(pbuf[q, pl.ds(0, L)] - m16)
    winv = 1.0 / z16
    plsc.subcore_barrier()

    def start_small(k, sb):
        pltpu.async_copy(esrc2.at[w * CH + k], idx_bufs[sb], smsems[sb])
        pltpu.async_copy(scores.at[pl.ds(w * EPW + k * C, C)], sc_bufs[sb],
                         smsems[sb])

    def wait_small(sb):
        pltpu.make_async_copy(esrc2.at[0], idx_bufs[sb], smsems[sb]).wait()
        pltpu.make_async_copy(scores.at[pl.ds(0, C)], sc_bufs[sb],
                              smsems[sb]).wait()

    def start_gather(sb, rb):
        pltpu.async_copy(psrc.at[idx_bufs[sb]], rows_bufs[rb], gsems[rb])

    def wait_gather(rb):
        pltpu.make_async_copy(psrc.at[idx_bufs[0]], rows_bufs[rb], gsems[rb]).wait()

    def start_scatter(k, rb):
        pltpu.async_copy(rows_bufs[rb], acc_shared.at[slab_d.at[k]], ssems[rb],
                         add=True)

    def wait_scatter(rb):
        pltpu.make_async_copy(rows_bufs[rb], acc_shared.at[slab_d.at[0]],
                              ssems[rb]).wait()

    def compute(sb, rb):
        rows = rows_bufs[rb]
        sc_c = sc_bufs[sb]

        for g in range(GPC):
            sc = sc_c[pl.ds(g * L, L)]
            sc_c[pl.ds(g * L, L)] = jnp.exp(sc - m16) * winv

        def group(g, carry):
            for i in range(L):
                e = g * L + i
                we = plsc.load_gather(sc_c, [jnp.broadcast_to(e, (L,))])
                for j in range(D // L):
                    rows[e, pl.ds(j * L, L)] = rows[e, pl.ds(j * L, L)] * we
            return carry

        lax.fori_loop(0, GPC, group, 0)

    start_small(0, 0)
    start_small(1, 1)
    wait_small(0)
    start_gather(0, 0)

    def trip(t, carry):
        for j in range(RB):
            k = RB * t + j
            rb = j
            nrb = (j + 1) % RB
            nsb = (j + 1) % SB
            psb = (j + 2) % SB

            def turn(k=k, rb=rb, nrb=nrb, nsb=nsb, psb=psb, j=j):
                @pl.when(k + 1 < CH)
                def _():
                    wait_small(nsb)

                    @pl.when(k >= 2)
                    def _():
                        wait_scatter(nrb)

                    start_gather(nsb, nrb)

                wait_gather(rb)
                compute(j % SB, rb)
                start_scatter(k, rb)

                @pl.when(k + 2 < CH)
                def _():
                    start_small(k + 2, psb)

            if j == 0:
                turn()
            else:
                pl.when(k < CH)(turn)
        return carry

    lax.fori_loop(0, -(-CH // RB), trip, 0)

    for rb in range(RB):
        wait_scatter(rb)
    plsc.subcore_barrier()

    r0 = s * ROWS_PER_SUB
    pltpu.sync_copy(
        acc_shared.at[pl.ds(r0, ROWS_PER_SUB)],
        part_out.at[c, pl.ds(r0, ROWS_PER_SUB)],
    )


def _aggregate(psrc, esrc2, edst3, scores, parts):
    zeros_blk = jnp.zeros((ROWS_PER_SUB, D), jnp.float32)

    def body(psrc_r, esrc_r, edst_r, scores_r, parts_r, zeros_r, out_r,
             slab_d, r0, r1, r2, i0, i1, i2, s0, s1, s2, pbuf, acc_shared,
             g0, g1, g2, t0, t1, t2, m0, m1, m2):
        _agg_kernel(psrc_r, esrc_r, edst_r, scores_r, parts_r, zeros_r, out_r,
                    slab_d, [r0, r1, r2], [i0, i1, i2], [s0, s1, s2],
                    pbuf, acc_shared, [g0, g1, g2], [t0, t1, t2],
                    [m0, m1, m2])

    return pl.kernel(
        body,
        out_type=jax.ShapeDtypeStruct((NC, N, D), jnp.float32),
        mesh=_mesh(),
        compiler_params=pltpu.CompilerParams(
            needs_layout_passes=False, use_tc_tiling_on_sc=False),
        scratch_types=[
            pltpu.VMEM((CH, C), jnp.int32),
            pltpu.VMEM((C, D), jnp.float32),
            pltpu.VMEM((C, D), jnp.float32),
            pltpu.VMEM((C, D), jnp.float32),
            pltpu.VMEM((C,), jnp.int32),
            pltpu.VMEM((C,), jnp.int32),
            pltpu.VMEM((C,), jnp.int32),
            pltpu.VMEM((C,), jnp.float32),
            pltpu.VMEM((C,), jnp.float32),
            pltpu.VMEM((C,), jnp.float32),
            pltpu.VMEM((NW, 2 * L), jnp.float32),
            pltpu.VMEM_SHARED((N, D), jnp.float32),
            pltpu.SemaphoreType.DMA,
            pltpu.SemaphoreType.DMA,
            pltpu.SemaphoreType.DMA,
            pltpu.SemaphoreType.DMA,
            pltpu.SemaphoreType.DMA,
            pltpu.SemaphoreType.DMA,
            pltpu.SemaphoreType.DMA,
            pltpu.SemaphoreType.DMA,
            pltpu.SemaphoreType.DMA,
        ],
    )(psrc, esrc2, edst3, scores, parts, zeros_blk)


def _combine_body(p_ref, o_ref):
    o_ref[...] = p_ref[0] + p_ref[1]


def _combine(partials):
    return pl.pallas_call(
        _combine_body,
        out_shape=jax.ShapeDtypeStruct((N, D), jnp.float32),
    )(partials)


def kernel(src, dst, edge_index, W_src, b_src, W_dst, b_dst):
    esrc3 = edge_index[0].reshape(NW, CH, C)
    esrc2 = edge_index[0].reshape(NW * CH, C)
    edst3 = edge_index[1].reshape(NW, CH, C)
    psrc, pdst = _project(src, dst, W_src, b_src, W_dst, b_dst)
    scores, parts = _edge_scores(psrc, pdst, esrc3, edst3)
    partials = _aggregate(psrc, esrc2, edst3, scores, parts)
    return _combine(partials)


# speedup vs baseline: 1.2281x; 1.0262x over previous
import functools

import jax
import jax.numpy as jnp
from jax import lax
from jax.experimental import pallas as pl
from jax.experimental.pallas import tpu as pltpu
from jax.experimental.pallas import tpu_sc as plsc

N = 10000
E = 320000
D = 128
L = 16
NC = 2
NS = 16
NW = NC * NS
EPW = E // NW
C = 80
CH = EPW // C
GPC = C // L
ROWS_PER_SUB = N // NS

_mesh = functools.partial(
    plsc.VectorSubcoreMesh,
    core_axis_name="c", subcore_axis_name="s", num_cores=NC, num_subcores=NS,
)


def _worker_id():
    return lax.axis_index("s") * NC + lax.axis_index("c")


def _proj_body(src_ref, dst_ref, ws_ref, bs_ref, wd_ref, bd_ref, ps_ref, pd_ref):
    ps_ref[...] = (
        jnp.dot(src_ref[...], ws_ref[...], preferred_element_type=jnp.float32)
        + bs_ref[...]
    )
    pd_ref[...] = (
        jnp.dot(dst_ref[...], wd_ref[...], preferred_element_type=jnp.float32)
        + bd_ref[...]
    )


def _project(src, dst, W_src, b_src, W_dst, b_dst):
    blk = 1000
    grid = N // blk
    return pl.pallas_call(
        _proj_body,
        grid=(grid,),
        in_specs=[
            pl.BlockSpec((blk, D), lambda i: (i, 0)),
            pl.BlockSpec((blk, D), lambda i: (i, 0)),
            pl.BlockSpec((D, D), lambda i: (0, 0)),
            pl.BlockSpec((1, D), lambda i: (0, 0)),
            pl.BlockSpec((D, D), lambda i: (0, 0)),
            pl.BlockSpec((1, D), lambda i: (0, 0)),
        ],
        out_specs=[
            pl.BlockSpec((blk, D), lambda i: (i, 0)),
            pl.BlockSpec((blk, D), lambda i: (i, 0)),
        ],
        out_shape=[
            jax.ShapeDtypeStruct((N, D), jnp.float32),
            jax.ShapeDtypeStruct((N, D), jnp.float32),
        ],
    )(src, dst, W_src, b_src.reshape(1, D), W_dst, b_dst.reshape(1, D))


SNB = 4


def _scores_kernel(psrc, pdst, esrc3, edst3, scores_out,
                   slab_s, slab_d, rs_bufs, rd_bufs,
                   scores_slab, stage, sems):
    w = _worker_id()
    lane_col = lax.broadcasted_iota(jnp.int32, (L,), 0) * L

    pltpu.sync_copy(esrc3.at[w], slab_s)
    pltpu.sync_copy(edst3.at[w], slab_d)

    def start_gather(k, b):
        pltpu.async_copy(psrc.at[slab_s.at[k]], rs_bufs[b], sems[b])
        pltpu.async_copy(pdst.at[slab_d.at[k]], rd_bufs[b], sems[b])

    def wait_gather(b):
        pltpu.make_async_copy(psrc.at[slab_s.at[0]], rs_bufs[b], sems[b]).wait()
        pltpu.make_async_copy(pdst.at[slab_d.at[0]], rd_bufs[b], sems[b]).wait()

    def compute(k, b):
        s_rows, d_rows = rs_bufs[b], rd_bufs[b]

        def group(g, carry):
            for i in range(L):
                e = g * L + i
                acc = s_rows[e, pl.ds(0, L)] * d_rows[e, pl.ds(0, L)]
                for j in range(1, D // L):
                    acc = acc + (s_rows[e, pl.ds(j * L, L)]
                                 * d_rows[e, pl.ds(j * L, L)])
                plsc.store_scatter(stage, [lane_col + i], acc)
            sums = stage[pl.ds(0, L)]
            for j in range(1, L):
                sums = sums + stage[pl.ds(j * L, L)]
            scores_slab[pl.ds(k * C + g * L, L)] = sums
            return carry

        lax.fori_loop(0, GPC, group, 0)

    for b in range(SNB - 1):
        start_gather(b, b)

    def quad(t, carry):
        for j in range(SNB):
            k = SNB * t + j

            def turn(k=k, b=j):
                wait_gather(b)
                compute(k, b)

                @pl.when(k + (SNB - 1) < CH)
                def _():
                    start_gather(k + (SNB - 1), (b + SNB - 1) % SNB)

            if j == 0:
                turn()
            else:
                pl.when(k < CH)(turn)
        return carry

    lax.fori_loop(0, -(-CH // SNB), quad, 0)

    pltpu.sync_copy(scores_slab, scores_out.at[pl.ds(w * EPW, EPW)])


def _edge_scores(psrc, pdst, esrc3, edst3):
    def body(psrc_r, pdst_r, esrc_r, edst_r, scores_r,
             slab_s, slab_d, rs0, rs1, rs2, rs3, rd0, rd1, rd2, rd3,
             scores_slab, stage, m0, m1, m2, m3):
        _scores_kernel(psrc_r, pdst_r, esrc_r, edst_r, scores_r,
                       slab_s, slab_d, [rs0, rs1, rs2, rs3],
                       [rd0, rd1, rd2, rd3], scores_slab, stage,
                       [m0, m1, m2, m3])

    return pl.kernel(
        body,
        out_type=jax.ShapeDtypeStruct((E,), jnp.float32),
        mesh=_mesh(),
        compiler_params=pltpu.CompilerParams(needs_layout_passes=False),
        scratch_types=[
            pltpu.VMEM((CH, C), jnp.int32),
            pltpu.VMEM((CH, C), jnp.int32),
            pltpu.VMEM((C, D), jnp.float32),
            pltpu.VMEM((C, D), jnp.float32),
            pltpu.VMEM((C, D), jnp.float32),
            pltpu.VMEM((C, D), jnp.float32),
            pltpu.VMEM((C, D), jnp.float32),
            pltpu.VMEM((C, D), jnp.float32),
            pltpu.VMEM((C, D), jnp.float32),
            pltpu.VMEM((C, D), jnp.float32),
            pltpu.VMEM((EPW,), jnp.float32),
            pltpu.VMEM((L * L,), jnp.float32),
            pltpu.SemaphoreType.DMA,
            pltpu.SemaphoreType.DMA,
            pltpu.SemaphoreType.DMA,
            pltpu.SemaphoreType.DMA,
        ],
    )(psrc, pdst, esrc3, edst3)


def _softmax_body(s_ref, o_ref):
    x = s_ref[...]
    m = jnp.max(x)
    e = jnp.exp(x - m)
    o_ref[...] = e / jnp.sum(e)


def _softmax(scores):
    s2 = scores.reshape(E // D, D)
    w = pl.pallas_call(
        _softmax_body,
        out_shape=jax.ShapeDtypeStruct((E // D, D), jnp.float32),
    )(s2)
    return w.reshape(E)


RB = 3
SB = 3


def _agg_kernel(psrc, esrc2, edst3, wts, zeros_blk, part_out,
                slab_d, rows_bufs, idx_bufs, sc_bufs, acc_shared,
                gsems, ssems, smsems):
    c = lax.axis_index("c")
    s = lax.axis_index("s")
    w = s * NC + c

    pltpu.sync_copy(zeros_blk, acc_shared.at[pl.ds(s * ROWS_PER_SUB, ROWS_PER_SUB)])
    pltpu.sync_copy(edst3.at[w], slab_d)
    plsc.subcore_barrier()

    def start_small(k, sb):
        pltpu.async_copy(esrc2.at[w * CH + k], idx_bufs[sb], smsems[sb])
        pltpu.async_copy(wts.at[pl.ds(w * EPW + k * C, C)], sc_bufs[sb],
                         smsems[sb])

    def wait_small(sb):
        pltpu.make_async_copy(esrc2.at[0], idx_bufs[sb], smsems[sb]).wait()
        pltpu.make_async_copy(wts.at[pl.ds(0, C)], sc_bufs[sb],
                              smsems[sb]).wait()

    def start_gather(sb, rb):
        pltpu.async_copy(psrc.at[idx_bufs[sb]], rows_bufs[rb], gsems[rb])

    def wait_gather(rb):
        pltpu.make_async_copy(psrc.at[idx_bufs[0]], rows_bufs[rb], gsems[rb]).wait()

    def start_scatter(k, rb):
        pltpu.async_copy(rows_bufs[rb], acc_shared.at[slab_d.at[k]], ssems[rb],
                         add=True)

    def wait_scatter(rb):
        pltpu.make_async_copy(rows_bufs[rb], acc_shared.at[slab_d.at[0]],
                              ssems[rb]).wait()

    def compute(sb, rb):
        rows = rows_bufs[rb]
        sc_c = sc_bufs[sb]

        def group(g, carry):
            for i in range(L):
                e = g * L + i
                we = plsc.load_gather(sc_c, [jnp.broadcast_to(e, (L,))])
                for j in range(D // L):
                    rows[e, pl.ds(j * L, L)] = rows[e, pl.ds(j * L, L)] * we
            return carry

        lax.fori_loop(0, GPC, group, 0)

    start_small(0, 0)
    start_small(1, 1)
    wait_small(0)
    start_gather(0, 0)

    def trip(t, carry):
        for j in range(RB):
            k = RB * t + j
            rb = j
            nrb = (j + 1) % RB
            nsb = (j + 1) % SB
            psb = (j + 2) % SB

            def turn(k=k, rb=rb, nrb=nrb, nsb=nsb, psb=psb, j=j):
                @pl.when(k + 1 < CH)
                def _():
                    wait_small(nsb)

                    @pl.when(k >= 2)
                    def _():
                        wait_scatter(nrb)

                    start_gather(nsb, nrb)

                wait_gather(rb)
                compute(j % SB, rb)
                start_scatter(k, rb)

                @pl.when(k + 2 < CH)
                def _():
                    start_small(k + 2, psb)

            if j == 0:
                turn()
            else:
                pl.when(k < CH)(turn)
        return carry

    lax.fori_loop(0, -(-CH // RB), trip, 0)

    for rb in range(RB):
        wait_scatter(rb)
    plsc.subcore_barrier()

    r0 = s * ROWS_PER_SUB
    pltpu.sync_copy(
        acc_shared.at[pl.ds(r0, ROWS_PER_SUB)],
        part_out.at[c, pl.ds(r0, ROWS_PER_SUB)],
    )


def _aggregate(psrc, esrc2, edst3, wts):
    zeros_blk = jnp.zeros((ROWS_PER_SUB, D), jnp.float32)

    def body(psrc_r, esrc_r, edst_r, wts_r, zeros_r, out_r,
             slab_d, r0, r1, r2, i0, i1, i2, s0, s1, s2, acc_shared,
             g0, g1, g2, t0, t1, t2, m0, m1, m2):
        _agg_kernel(psrc_r, esrc_r, edst_r, wts_r, zeros_r, out_r,
                    slab_d, [r0, r1, r2], [i0, i1, i2], [s0, s1, s2],
                    acc_shared, [g0, g1, g2], [t0, t1, t2],
                    [m0, m1, m2])

    return pl.kernel(
        body,
        out_type=jax.ShapeDtypeStruct((NC, N, D), jnp.float32),
        mesh=_mesh(),
        compiler_params=pltpu.CompilerParams(
            needs_layout_passes=False, use_tc_tiling_on_sc=False),
        scratch_types=[
            pltpu.VMEM((CH, C), jnp.int32),
            pltpu.VMEM((C, D), jnp.float32),
            pltpu.VMEM((C, D), jnp.float32),
            pltpu.VMEM((C, D), jnp.float32),
            pltpu.VMEM((C,), jnp.int32),
            pltpu.VMEM((C,), jnp.int32),
            pltpu.VMEM((C,), jnp.int32),
            pltpu.VMEM((C,), jnp.float32),
            pltpu.VMEM((C,), jnp.float32),
            pltpu.VMEM((C,), jnp.float32),
            pltpu.VMEM_SHARED((N, D), jnp.float32),
            pltpu.SemaphoreType.DMA,
            pltpu.SemaphoreType.DMA,
            pltpu.SemaphoreType.DMA,
            pltpu.SemaphoreType.DMA,
            pltpu.SemaphoreType.DMA,
            pltpu.SemaphoreType.DMA,
            pltpu.SemaphoreType.DMA,
            pltpu.SemaphoreType.DMA,
            pltpu.SemaphoreType.DMA,
        ],
    )(psrc, esrc2, edst3, wts, zeros_blk)


def _combine_body(p_ref, o_ref):
    o_ref[...] = p_ref[0] + p_ref[1]


def _combine(partials):
    return pl.pallas_call(
        _combine_body,
        out_shape=jax.ShapeDtypeStruct((N, D), jnp.float32),
    )(partials)


def kernel(src, dst, edge_index, W_src, b_src, W_dst, b_dst):
    esrc3 = edge_index[0].reshape(NW, CH, C)
    esrc2 = edge_index[0].reshape(NW * CH, C)
    edst3 = edge_index[1].reshape(NW, CH, C)
    psrc, pdst = _project(src, dst, W_src, b_src, W_dst, b_dst)
    scores = _edge_scores(psrc, pdst, esrc3, edst3)
    wts = _softmax(scores)
    partials = _aggregate(psrc, esrc2, edst3, wts)
    return _combine(partials)
